# baseline (speedup 1.0000x reference)
import functools

import jax
import jax.numpy as jnp
from jax import lax
from jax.experimental import pallas as pl
from jax.experimental.pallas import tpu as pltpu
from jax.experimental.pallas import tpu_sc as plsc

N, E, D, H, F = 10000, 320000, 128, 8, 16
HF = H * F
RW = HF + 16
BN = 2000
GRID = N // BN

_DOT = dict(preferred_element_type=jnp.float32, precision=lax.Precision.HIGHEST)


def _ehf():
    r = lax.broadcasted_iota(jnp.int32, (H, HF), 1) // F
    c = lax.broadcasted_iota(jnp.int32, (H, HF), 0)
    return (r == c).astype(jnp.float32)


def _pre_body(x_ref, wp_ref, asrc_ref, atrg_ref, a_ref, st_ref, bm_ref):
    x = x_ref[...]
    proj = lax.dot_general(x, wp_ref[...], (((1,), (1,)), ((), ())), **_DOT)
    ehf = _ehf()
    ss = lax.dot_general(proj * asrc_ref[...], ehf, (((1,), (1,)), ((), ())), **_DOT)
    st = lax.dot_general(proj * atrg_ref[...], ehf, (((1,), (1,)), ((), ())), **_DOT)
    zpad = jnp.zeros((x.shape[0], 8), jnp.float32)
    a_ref[...] = jnp.concatenate([proj, ss, zpad], axis=1)
    st_ref[...] = jnp.concatenate([st, zpad], axis=1)
    bm = jnp.concatenate([jnp.max(ss, axis=0), jnp.max(st, axis=0),
                          jnp.full((112,), -1e30, jnp.float32)])
    bm_ref[...] = jnp.broadcast_to(bm.reshape(1, HF), (8, HF))


def _pre(x, W_proj, a_src128, a_trg128):
    return pl.pallas_call(
        _pre_body,
        grid=(GRID,),
        in_specs=[
            pl.BlockSpec((BN, D), lambda i: (i, 0)),
            pl.BlockSpec((HF, D), lambda i: (0, 0)),
            pl.BlockSpec((1, HF), lambda i: (0, 0)),
            pl.BlockSpec((1, HF), lambda i: (0, 0)),
        ],
        out_specs=[
            pl.BlockSpec((BN, RW), lambda i: (i, 0)),
            pl.BlockSpec((BN, 16), lambda i: (i, 0)),
            pl.BlockSpec((8, HF), lambda i: (i, 0)),
        ],
        out_shape=[
            jax.ShapeDtypeStruct((N, RW), jnp.float32),
            jax.ShapeDtypeStruct((N, 16), jnp.float32),
            jax.ShapeDtypeStruct((GRID * 8, HF), jnp.float32),
        ],
    )(x, W_proj, a_src128, a_trg128)


def _post_body(acc_ref, x_ref, wsk_ref, bias_ref, out_ref):
    num = acc_ref[0, :, :HF] + acc_ref[1, :, :HF]
    den = acc_ref[0, :, HF:HF + H] + acc_ref[1, :, HF:HF + H]
    recip = 1.0 / (den + 1e-16)
    recip128 = lax.dot_general(recip, _ehf(), (((1,), (0,)), ((), ())), **_DOT)
    skip = lax.dot_general(x_ref[...], wsk_ref[...], (((1,), (1,)), ((), ())), **_DOT)
    o = num * recip128 + skip + bias_ref[...]
    out_ref[...] = jnp.where(o > 0, o, jnp.exp(jnp.minimum(o, 0.0)) - 1.0)


def _post(acc2, x, W_skip, bias128):
    return pl.pallas_call(
        _post_body,
        grid=(GRID,),
        in_specs=[
            pl.BlockSpec((2, BN, RW), lambda i: (0, i, 0)),
            pl.BlockSpec((BN, D), lambda i: (i, 0)),
            pl.BlockSpec((HF, D), lambda i: (0, 0)),
            pl.BlockSpec((1, HF), lambda i: (0, 0)),
        ],
        out_specs=pl.BlockSpec((BN, HF), lambda i: (i, 0)),
        out_shape=jax.ShapeDtypeStruct((N, HF), jnp.float32),
    )(acc2, x, W_skip, bias128)


NC, NS = 2, 16
NW = NC * NS
EPW = E // NW
K = 40
CB = 50
NCH = EPW // (K * CB)
NPAIR = CB // 2
NROW = E // K
RPW = EPW // K
NP = 10240
RPS = NP // NS
_SC_MESH = plsc.VectorSubcoreMesh(core_axis_name="c", subcore_axis_name="s")


def _edge_body(a_hbm, st_hbm, src_hbm, trg_hbm, p_hbm, cm_hbm, acc_hbm,
               acc_sh, cm_v, src_big, trg_big, p_big,
               a0, a1, st0, st1, out0, out1,
               sem_a0, sem_a1, sem_st0, sem_st1, sem_sc0, sem_sc1):
    cid = lax.axis_index("c")
    sid = lax.axis_index("s")
    zvec = jnp.zeros((16,), jnp.float32)

    def zrow(r, carry):
        for jj in range(RW // 16):
            out0[r, pl.ds(jj * 16, 16)] = zvec
        return carry

    lax.fori_loop(0, K, zrow, 0)
    for jj in range(RPS // K):
        pltpu.sync_copy(out0, acc_sh.at[pl.ds(sid * RPS + jj * K, K)])
    pltpu.sync_copy(cm_hbm, cm_v)
    plsc.subcore_barrier()

    cvec = cm_v[0, :]
    mvec = cm_v[1, :]
    wrb = (cid * NS + sid) * RPW

    def issue(j, a_r, st_r, sa, ss_):
        pltpu.async_copy(a_hbm.at[src_big.at[j]], a_r, sa)
        pltpu.async_copy(st_hbm.at[trg_big.at[j]], st_r, ss_)

    def wait_g(a_r, st_r, sa, ss_):
        pltpu.make_async_copy(a_hbm.at[src_big.at[0]], a_r, sa).wait()
        pltpu.make_async_copy(st_hbm.at[trg_big.at[0]], st_r, ss_).wait()

    def scat(j, out_r, sc):
        pltpu.async_copy(out_r, acc_sh.at[trg_big.at[j]], sc, add=True)

    def wait_sc(out_r, sc):
        pltpu.make_async_copy(out_r, acc_sh.at[trg_big.at[0]], sc).wait()

    def compute(j, a_r, st_r, out_r):
        jv = jnp.full((16,), j, jnp.int32)

        @plsc.parallel_loop(0, K, unroll=8)
        def edge(e):
            ev = jnp.full((16,), e, jnp.int32)
            pe = plsc.load_gather(p_big, [jv, ev])
            s = a_r[e, pl.ds(HF, 16)] + st_r[e, pl.ds(0, 16)] + pe * cvec
            s = jnp.where(s > 0, s, 0.2 * s)
            w = jnp.exp(s - mvec)
            out_r[e, pl.ds(HF, 16)] = w
            for h in range(H):
                out_r[e, pl.ds(h * 16, 16)] = a_r[e, pl.ds(h * 16, 16)] * w[h]

    for c in range(NCH):
        if c > 0:
            wait_sc(out0, sem_sc0)
            wait_sc(out1, sem_sc1)
        r0 = wrb + c * CB
        pltpu.sync_copy(src_hbm.at[pl.ds(r0, CB)], src_big)
        pltpu.sync_copy(trg_hbm.at[pl.ds(r0, CB)], trg_big)
        pltpu.sync_copy(p_hbm.at[pl.ds(r0, CB)], p_big)
        issue(0, a0, st0, sem_a0, sem_st0)

        def pair(pp, carry):
            j0 = 2 * pp
            j1 = j0 + 1
            wait_g(a0, st0, sem_a0, sem_st0)
            issue(j1, a1, st1, sem_a1, sem_st1)

            @pl.when(pp >= 1)
            def _():
                wait_sc(out0, sem_sc0)

            compute(j0, a0, st0, out0)
            scat(j0, out0, sem_sc0)
            wait_g(a1, st1, sem_a1, sem_st1)

            @pl.when(pp <= NPAIR - 2)
            def _():
                issue(j1 + 1, a0, st0, sem_a0, sem_st0)

            @pl.when(pp >= 1)
            def _():
                wait_sc(out1, sem_sc1)

            compute(j1, a1, st1, out1)
            scat(j1, out1, sem_sc1)
            return carry

        lax.fori_loop(0, NPAIR, pair, 0)

    wait_sc(out0, sem_sc0)
    wait_sc(out1, sem_sc1)
    plsc.subcore_barrier()
    for jj in range(RPS // K):
        r0 = sid * RPS + jj * K
        pltpu.sync_copy(acc_sh.at[pl.ds(r0, K)], out0)
        pltpu.sync_copy(out0, acc_hbm.at[cid, pl.ds(r0, K)])


_edge_sc = pl.kernel(
    _edge_body,
    out_type=jax.ShapeDtypeStruct((NC, NP, RW), jnp.float32),
    mesh=_SC_MESH,
    scratch_types=[
        pltpu.VMEM_SHARED((NP, RW), jnp.float32),
        pltpu.VMEM((2, 16), jnp.float32),
        pltpu.VMEM((CB, K), jnp.int32),
        pltpu.VMEM((CB, K), jnp.int32),
        pltpu.VMEM((CB, K), jnp.float32),
        pltpu.VMEM((K, RW), jnp.float32),
        pltpu.VMEM((K, RW), jnp.float32),
        pltpu.VMEM((K, 16), jnp.float32),
        pltpu.VMEM((K, 16), jnp.float32),
        pltpu.VMEM((K, RW), jnp.float32),
        pltpu.VMEM((K, RW), jnp.float32),
        pltpu.SemaphoreType.DMA,
        pltpu.SemaphoreType.DMA,
        pltpu.SemaphoreType.DMA,
        pltpu.SemaphoreType.DMA,
        pltpu.SemaphoreType.DMA,
        pltpu.SemaphoreType.DMA,
    ],
    compiler_params=pltpu.CompilerParams(use_tc_tiling_on_sc=False,
                                         needs_layout_passes=False),
)


def _edge_stage_jax(a_arr, st_arr, src, trg, p, c, m):
    proj = a_arr[:, :HF]
    ss = a_arr[:, HF:HF + H]
    st = st_arr[:, :H]
    s = ss[src] + st[trg] + p[:, None] * c[None, :]
    s = jnp.where(s > 0, s, 0.2 * s)
    w = jnp.exp(s - m[None, :])
    den = jax.ops.segment_sum(w, trg, num_segments=N)
    num = jax.ops.segment_sum(
        (w[:, :, None] * proj[src].reshape(E, H, F)).reshape(E, HF), trg,
        num_segments=N)
    part = jnp.concatenate([num, den, jnp.zeros((N, 8), jnp.float32)], axis=1)
    return jnp.stack([part, jnp.zeros_like(part)])


def kernel(x, edge_index, edge_prob, W_proj, W_tp, a_src, a_trg, a_tp, W_skip, bias):
    a_src128 = a_src.reshape(1, HF)
    a_trg128 = a_trg.reshape(1, HF)
    bias128 = bias.reshape(1, HF)
    a_arr, st_arr, bm = _pre(x, W_proj, a_src128, a_trg128)
    c = (W_tp[:, 0].reshape(H, F) * a_tp[0]).sum(-1)
    m = bm[:, :H].max(0) + bm[:, H:2 * H].max(0) + jnp.maximum(c, 0.0)
    src = edge_index[0]
    trg = edge_index[1]
    p = edge_prob[:, 0]
    cm = jnp.stack([
        jnp.concatenate([c, jnp.zeros((8,), jnp.float32)]),
        jnp.concatenate([m, jnp.full((8,), 1e9, jnp.float32)]),
    ])
    acc2 = _edge_sc(a_arr, st_arr, src.reshape(NROW, K), trg.reshape(NROW, K),
                    p.reshape(NROW, K), cm)
    out = _post(acc2, x, W_skip, bias128)
    return (out, edge_index, edge_prob)


# speedup vs baseline: 1.1885x; 1.1885x over previous
import functools

import jax
import jax.numpy as jnp
from jax import lax
from jax.experimental import pallas as pl
from jax.experimental.pallas import tpu as pltpu
from jax.experimental.pallas import tpu_sc as plsc

N, E, D, H, F = 10000, 320000, 128, 8, 16
HF = H * F
RW = HF + 16
BN = 2000
GRID = N // BN

_DOT = dict(preferred_element_type=jnp.float32, precision=lax.Precision.HIGHEST)


def _ehf():
    r = lax.broadcasted_iota(jnp.int32, (H, HF), 1) // F
    c = lax.broadcasted_iota(jnp.int32, (H, HF), 0)
    return (r == c).astype(jnp.float32)


def _pre_body(x_ref, wp_ref, asrc_ref, atrg_ref, a_ref, st_ref, bm_ref):
    x = x_ref[...]
    proj = lax.dot_general(x, wp_ref[...], (((1,), (1,)), ((), ())), **_DOT)
    ehf = _ehf()
    ss = lax.dot_general(proj * asrc_ref[...], ehf, (((1,), (1,)), ((), ())), **_DOT)
    st = lax.dot_general(proj * atrg_ref[...], ehf, (((1,), (1,)), ((), ())), **_DOT)
    zpad = jnp.zeros((x.shape[0], 8), jnp.float32)
    a_ref[...] = jnp.concatenate([proj, ss, zpad], axis=1)
    st_ref[...] = jnp.concatenate([st, zpad], axis=1)
    bm = jnp.concatenate([jnp.max(ss, axis=0), jnp.max(st, axis=0),
                          jnp.full((112,), -1e30, jnp.float32)])
    bm_ref[...] = jnp.broadcast_to(bm.reshape(1, HF), (8, HF))


def _pre(x, W_proj, a_src128, a_trg128):
    return pl.pallas_call(
        _pre_body,
        grid=(GRID,),
        in_specs=[
            pl.BlockSpec((BN, D), lambda i: (i, 0)),
            pl.BlockSpec((HF, D), lambda i: (0, 0)),
            pl.BlockSpec((1, HF), lambda i: (0, 0)),
            pl.BlockSpec((1, HF), lambda i: (0, 0)),
        ],
        out_specs=[
            pl.BlockSpec((BN, RW), lambda i: (i, 0)),
            pl.BlockSpec((BN, 16), lambda i: (i, 0)),
            pl.BlockSpec((8, HF), lambda i: (i, 0)),
        ],
        out_shape=[
            jax.ShapeDtypeStruct((N, RW), jnp.float32),
            jax.ShapeDtypeStruct((N, 16), jnp.float32),
            jax.ShapeDtypeStruct((GRID * 8, HF), jnp.float32),
        ],
    )(x, W_proj, a_src128, a_trg128)


def _post_body(acc_ref, x_ref, wsk_ref, bias_ref, out_ref):
    num = acc_ref[0, :, :HF] + acc_ref[1, :, :HF]
    den = acc_ref[0, :, HF:HF + H] + acc_ref[1, :, HF:HF + H]
    recip = 1.0 / (den + 1e-16)
    recip128 = lax.dot_general(recip, _ehf(), (((1,), (0,)), ((), ())), **_DOT)
    skip = lax.dot_general(x_ref[...], wsk_ref[...], (((1,), (1,)), ((), ())), **_DOT)
    o = num * recip128 + skip + bias_ref[...]
    out_ref[...] = jnp.where(o > 0, o, jnp.exp(jnp.minimum(o, 0.0)) - 1.0)


def _post(acc2, x, W_skip, bias128):
    return pl.pallas_call(
        _post_body,
        grid=(GRID,),
        in_specs=[
            pl.BlockSpec((2, BN, 136), lambda i: (0, i, 0)),
            pl.BlockSpec((BN, D), lambda i: (i, 0)),
            pl.BlockSpec((HF, D), lambda i: (0, 0)),
            pl.BlockSpec((1, HF), lambda i: (0, 0)),
        ],
        out_specs=pl.BlockSpec((BN, HF), lambda i: (i, 0)),
        out_shape=jax.ShapeDtypeStruct((N, HF), jnp.float32),
    )(acc2, x, W_skip, bias128)


NC, NS = 2, 16
NW = NC * NS
EPW = E // NW
K = 40
CB = 50
NCH = EPW // (K * CB)
NSUP = 12
NROW = E // K
RPW = EPW // K
AW = HF + H
NP = 10112
RPS = NP // NS
_SC_MESH = plsc.VectorSubcoreMesh(core_axis_name="c", subcore_axis_name="s")


def _edge_body(a_hbm, st_hbm, src_hbm, trg_hbm, p_hbm, cm_hbm, acc_hbm,
               acc_sh, cm_v, src_big, trg_big, p_big,
               ax0, ax1, ay0, ay1, stx0, stx1, sty0, sty1, out0, out1,
               sem_ax0, sem_ax1, sem_ay0, sem_ay1,
               sem_sx0, sem_sx1, sem_sy0, sem_sy1, sem_sc0, sem_sc1):
    cid = lax.axis_index("c")
    sid = lax.axis_index("s")
    zvec = jnp.zeros((16,), jnp.float32)

    def zrow(r, carry):
        for jj in range(HF // 16):
            out0[r, pl.ds(jj * 16, 16)] = zvec
        out0[r, pl.ds(AW - 16, 16)] = zvec
        return carry

    lax.fori_loop(0, K, zrow, 0)
    for jj in range(15):
        pltpu.sync_copy(out0, acc_sh.at[pl.ds(sid * RPS + jj * K, K)])
    pltpu.sync_copy(out0.at[pl.ds(0, 32)],
                    acc_sh.at[pl.ds(sid * RPS + 600, 32)])
    pltpu.sync_copy(cm_hbm, cm_v)
    plsc.subcore_barrier()

    cvec = cm_v[0, :]
    mvec = cm_v[1, :]
    widx = jnp.arange(16, dtype=jnp.int32) + HF
    wmask = jnp.arange(16, dtype=jnp.int32) < H
    wrb = (cid * NS + sid) * RPW

    def issue(j, a_r, st_r, sa, ss_):
        pltpu.async_copy(a_hbm.at[src_big.at[j]], a_r, sa)
        pltpu.async_copy(st_hbm.at[trg_big.at[j]], st_r, ss_)

    def wait_g(a_r, st_r, sa, ss_):
        pltpu.make_async_copy(a_hbm.at[src_big.at[0]], a_r, sa).wait()
        pltpu.make_async_copy(st_hbm.at[trg_big.at[0]], st_r, ss_).wait()

    def scat(j, out_r, sc):
        pltpu.async_copy(out_r, acc_sh.at[trg_big.at[j]], sc, add=True)

    def wait_sc(out_r, sc):
        pltpu.make_async_copy(out_r, acc_sh.at[trg_big.at[0]], sc).wait()

    def compute(j, a_r, st_r, out_r):
        jv = jnp.full((16,), j, jnp.int32)

        @plsc.parallel_loop(0, K, unroll=4)
        def edge(e):
            ev = jnp.full((16,), e, jnp.int32)
            pe = plsc.load_gather(p_big, [jv, ev])
            s = a_r[e, pl.ds(HF, 16)] + st_r[e, pl.ds(0, 16)] + pe * cvec
            s = jnp.where(s > 0, s, 0.2 * s)
            w = jnp.exp(s - mvec)
            plsc.store_scatter(out_r, [ev, widx], w, mask=wmask)
            for h in range(H):
                out_r[e, pl.ds(h * 16, 16)] = a_r[e, pl.ds(h * 16, 16)] * w[h]

    def chunk(c, carry):
        @pl.when(c > 0)
        def _():
            wait_sc(out0, sem_sc0)
            wait_sc(out1, sem_sc1)

        r0 = wrb + c * CB
        pltpu.sync_copy(src_hbm.at[pl.ds(r0, CB)], src_big)
        pltpu.sync_copy(trg_hbm.at[pl.ds(r0, CB)], trg_big)
        pltpu.sync_copy(p_hbm.at[pl.ds(r0, CB)], p_big)
        issue(0, ax0, stx0, sem_ax0, sem_sx0)
        issue(1, ax1, stx1, sem_ax1, sem_sx1)

        def sup(q, carry2):
            j0 = 4 * q
            wait_g(ax0, stx0, sem_ax0, sem_sx0)
            wait_g(ax1, stx1, sem_ax1, sem_sx1)
            issue(j0 + 2, ay0, sty0, sem_ay0, sem_sy0)
            issue(j0 + 3, ay1, sty1, sem_ay1, sem_sy1)

            @pl.when(q >= 1)
            def _():
                wait_sc(out0, sem_sc0)
                wait_sc(out1, sem_sc1)

            compute(j0, ax0, stx0, out0)
            scat(j0, out0, sem_sc0)
            compute(j0 + 1, ax1, stx1, out1)
            scat(j0 + 1, out1, sem_sc1)
            wait_g(ay0, sty0, sem_ay0, sem_sy0)
            wait_g(ay1, sty1, sem_ay1, sem_sy1)
            issue(j0 + 4, ax0, stx0, sem_ax0, sem_sx0)
            issue(j0 + 5, ax1, stx1, sem_ax1, sem_sx1)
            wait_sc(out0, sem_sc0)
            wait_sc(out1, sem_sc1)
            compute(j0 + 2, ay0, sty0, out0)
            scat(j0 + 2, out0, sem_sc0)
            compute(j0 + 3, ay1, sty1, out1)
            scat(j0 + 3, out1, sem_sc1)
            return carry2

        lax.fori_loop(0, NSUP, sup, 0)
        wait_g(ax0, stx0, sem_ax0, sem_sx0)
        wait_g(ax1, stx1, sem_ax1, sem_sx1)
        wait_sc(out0, sem_sc0)
        wait_sc(out1, sem_sc1)
        compute(CB - 2, ax0, stx0, out0)
        scat(CB - 2, out0, sem_sc0)
        compute(CB - 1, ax1, stx1, out1)
        scat(CB - 1, out1, sem_sc1)
        return carry

    lax.fori_loop(0, NCH, chunk, 0)
    wait_sc(out0, sem_sc0)
    wait_sc(out1, sem_sc1)
    plsc.subcore_barrier()
    for jj in range(15):
        r0 = sid * RPS + jj * K
        pltpu.sync_copy(acc_sh.at[pl.ds(r0, K)], out0)
        pltpu.sync_copy(out0, acc_hbm.at[cid, pl.ds(r0, K)])
    r0 = sid * RPS + 600
    pltpu.sync_copy(acc_sh.at[pl.ds(r0, 32)], out0.at[pl.ds(0, 32)])
    pltpu.sync_copy(out0.at[pl.ds(0, 32)], acc_hbm.at[cid, pl.ds(r0, 32)])


_edge_sc = pl.kernel(
    _edge_body,
    out_type=jax.ShapeDtypeStruct((NC, NP, AW), jnp.float32),
    mesh=_SC_MESH,
    scratch_types=[
        pltpu.VMEM_SHARED((NP, AW), jnp.float32),
        pltpu.VMEM((2, 16), jnp.float32),
        pltpu.VMEM((CB, K), jnp.int32),
        pltpu.VMEM((CB, K), jnp.int32),
        pltpu.VMEM((CB, K), jnp.float32),
        pltpu.VMEM((K, RW), jnp.float32),
        pltpu.VMEM((K, RW), jnp.float32),
        pltpu.VMEM((K, RW), jnp.float32),
        pltpu.VMEM((K, RW), jnp.float32),
        pltpu.VMEM((K, 16), jnp.float32),
        pltpu.VMEM((K, 16), jnp.float32),
        pltpu.VMEM((K, 16), jnp.float32),
        pltpu.VMEM((K, 16), jnp.float32),
        pltpu.VMEM((K, AW), jnp.float32),
        pltpu.VMEM((K, AW), jnp.float32),
        pltpu.SemaphoreType.DMA,
        pltpu.SemaphoreType.DMA,
        pltpu.SemaphoreType.DMA,
        pltpu.SemaphoreType.DMA,
        pltpu.SemaphoreType.DMA,
        pltpu.SemaphoreType.DMA,
        pltpu.SemaphoreType.DMA,
        pltpu.SemaphoreType.DMA,
        pltpu.SemaphoreType.DMA,
        pltpu.SemaphoreType.DMA,
    ],
    compiler_params=pltpu.CompilerParams(use_tc_tiling_on_sc=False,
                                         needs_layout_passes=False),
)


def _edge_stage_jax(a_arr, st_arr, src, trg, p, c, m):
    proj = a_arr[:, :HF]
    ss = a_arr[:, HF:HF + H]
    st = st_arr[:, :H]
    s = ss[src] + st[trg] + p[:, None] * c[None, :]
    s = jnp.where(s > 0, s, 0.2 * s)
    w = jnp.exp(s - m[None, :])
    den = jax.ops.segment_sum(w, trg, num_segments=N)
    num = jax.ops.segment_sum(
        (w[:, :, None] * proj[src].reshape(E, H, F)).reshape(E, HF), trg,
        num_segments=N)
    part = jnp.concatenate([num, den, jnp.zeros((N, 8), jnp.float32)], axis=1)
    return jnp.stack([part, jnp.zeros_like(part)])


def kernel(x, edge_index, edge_prob, W_proj, W_tp, a_src, a_trg, a_tp, W_skip, bias):
    a_src128 = a_src.reshape(1, HF)
    a_trg128 = a_trg.reshape(1, HF)
    bias128 = bias.reshape(1, HF)
    a_arr, st_arr, bm = _pre(x, W_proj, a_src128, a_trg128)
    c = (W_tp[:, 0].reshape(H, F) * a_tp[0]).sum(-1)
    m = bm[:, :H].max(0) + bm[:, H:2 * H].max(0) + jnp.maximum(c, 0.0)
    src = edge_index[0]
    trg = edge_index[1]
    p = edge_prob[:, 0]
    cm = jnp.stack([
        jnp.concatenate([c, jnp.zeros((8,), jnp.float32)]),
        jnp.concatenate([m, jnp.full((8,), 1e9, jnp.float32)]),
    ])
    acc2 = _edge_sc(a_arr, st_arr, src.reshape(NROW, K), trg.reshape(NROW, K),
                    p.reshape(NROW, K), cm)
    out = _post(acc2, x, W_skip, bias128)
    return (out, edge_index, edge_prob)


# speedup vs baseline: 1.1893x; 1.0007x over previous
import functools

import jax
import jax.numpy as jnp
from jax import lax
from jax.experimental import pallas as pl
from jax.experimental.pallas import tpu as pltpu
from jax.experimental.pallas import tpu_sc as plsc

N, E, D, H, F = 10000, 320000, 128, 8, 16
HF = H * F
RW = HF + 16
BN = 2000
GRID = N // BN

_DOT = dict(preferred_element_type=jnp.float32, precision=lax.Precision.HIGHEST)


def _ehf():
    r = lax.broadcasted_iota(jnp.int32, (H, HF), 1) // F
    c = lax.broadcasted_iota(jnp.int32, (H, HF), 0)
    return (r == c).astype(jnp.float32)


def _pre_body(x_ref, wp_ref, asrc_ref, atrg_ref, a_ref, st_ref, bm_ref):
    x = x_ref[...]
    proj = lax.dot_general(x, wp_ref[...], (((1,), (1,)), ((), ())), **_DOT)
    ehf = _ehf()
    ss = lax.dot_general(proj * asrc_ref[...], ehf, (((1,), (1,)), ((), ())), **_DOT)
    st = lax.dot_general(proj * atrg_ref[...], ehf, (((1,), (1,)), ((), ())), **_DOT)
    zpad = jnp.zeros((x.shape[0], 8), jnp.float32)
    a_ref[...] = jnp.concatenate([proj, ss, zpad], axis=1)
    st_ref[...] = jnp.concatenate([st, zpad], axis=1)
    bm = jnp.concatenate([jnp.max(ss, axis=0), jnp.max(st, axis=0),
                          jnp.full((112,), -1e30, jnp.float32)])
    bm_ref[...] = jnp.broadcast_to(bm.reshape(1, HF), (8, HF))


def _pre(x, W_proj, a_src128, a_trg128):
    return pl.pallas_call(
        _pre_body,
        grid=(GRID,),
        in_specs=[
            pl.BlockSpec((BN, D), lambda i: (i, 0)),
            pl.BlockSpec((HF, D), lambda i: (0, 0)),
            pl.BlockSpec((1, HF), lambda i: (0, 0)),
            pl.BlockSpec((1, HF), lambda i: (0, 0)),
        ],
        out_specs=[
            pl.BlockSpec((BN, RW), lambda i: (i, 0)),
            pl.BlockSpec((BN, 16), lambda i: (i, 0)),
            pl.BlockSpec((8, HF), lambda i: (i, 0)),
        ],
        out_shape=[
            jax.ShapeDtypeStruct((N, RW), jnp.float32),
            jax.ShapeDtypeStruct((N, 16), jnp.float32),
            jax.ShapeDtypeStruct((GRID * 8, HF), jnp.float32),
        ],
    )(x, W_proj, a_src128, a_trg128)


def _post_body(acc_ref, x_ref, wsk_ref, bias_ref, out_ref):
    num = acc_ref[0, :, :HF] + acc_ref[1, :, :HF]
    den = acc_ref[0, :, HF:HF + H] + acc_ref[1, :, HF:HF + H]
    recip = 1.0 / (den + 1e-16)
    recip128 = lax.dot_general(recip, _ehf(), (((1,), (0,)), ((), ())), **_DOT)
    skip = lax.dot_general(x_ref[...], wsk_ref[...], (((1,), (1,)), ((), ())), **_DOT)
    o = num * recip128 + skip + bias_ref[...]
    out_ref[...] = jnp.where(o > 0, o, jnp.exp(jnp.minimum(o, 0.0)) - 1.0)


def _post(acc2, x, W_skip, bias128):
    return pl.pallas_call(
        _post_body,
        grid=(GRID,),
        in_specs=[
            pl.BlockSpec((2, BN, 136), lambda i: (0, i, 0)),
            pl.BlockSpec((BN, D), lambda i: (i, 0)),
            pl.BlockSpec((HF, D), lambda i: (0, 0)),
            pl.BlockSpec((1, HF), lambda i: (0, 0)),
        ],
        out_specs=pl.BlockSpec((BN, HF), lambda i: (i, 0)),
        out_shape=jax.ShapeDtypeStruct((N, HF), jnp.float32),
    )(acc2, x, W_skip, bias128)


NC, NS = 2, 16
NW = NC * NS
EPW = E // NW
K = 40
CB = 50
NCH = EPW // (K * CB)
NSUP = 12
NROW = E // K
RPW = EPW // K
AW = HF + H
NP = 10112
RPS = NP // NS
_SC_MESH = plsc.VectorSubcoreMesh(core_axis_name="c", subcore_axis_name="s")


def _edge_body(a_hbm, st_hbm, src_hbm, trg_hbm, p_hbm, cm_hbm, acc_hbm,
               acc_sh, cm_v, src_big, trg_big, p_big,
               ax0, ax1, ay0, ay1, stx0, stx1, sty0, sty1, out0, out1,
               sem_ax0, sem_ax1, sem_ay0, sem_ay1,
               sem_sx0, sem_sx1, sem_sy0, sem_sy1, sem_sc0, sem_sc1):
    cid = lax.axis_index("c")
    sid = lax.axis_index("s")
    zvec = jnp.zeros((16,), jnp.float32)

    def zrow(r, carry):
        for jj in range(HF // 16):
            out0[r, pl.ds(jj * 16, 16)] = zvec
        out0[r, pl.ds(AW - 16, 16)] = zvec
        return carry

    lax.fori_loop(0, K, zrow, 0)
    for jj in range(15):
        pltpu.async_copy(out0, acc_sh.at[pl.ds(sid * RPS + jj * K, K)], sem_sc0)
    pltpu.async_copy(out0.at[pl.ds(0, 32)],
                     acc_sh.at[pl.ds(sid * RPS + 600, 32)], sem_sc1)
    for jj in range(15):
        pltpu.make_async_copy(out0, acc_sh.at[pl.ds(sid * RPS + jj * K, K)],
                              sem_sc0).wait()
    pltpu.make_async_copy(out0.at[pl.ds(0, 32)],
                          acc_sh.at[pl.ds(sid * RPS + 600, 32)], sem_sc1).wait()
    pltpu.sync_copy(cm_hbm, cm_v)
    plsc.subcore_barrier()

    cvec = cm_v[0, :]
    mvec = cm_v[1, :]
    widx = jnp.arange(16, dtype=jnp.int32) + HF
    wmask = jnp.arange(16, dtype=jnp.int32) < H
    wrb = (cid * NS + sid) * RPW

    def issue(j, a_r, st_r, sa, ss_):
        pltpu.async_copy(a_hbm.at[src_big.at[j]], a_r, sa)
        pltpu.async_copy(st_hbm.at[trg_big.at[j]], st_r, ss_)

    def wait_g(a_r, st_r, sa, ss_):
        pltpu.make_async_copy(a_hbm.at[src_big.at[0]], a_r, sa).wait()
        pltpu.make_async_copy(st_hbm.at[trg_big.at[0]], st_r, ss_).wait()

    def scat(j, out_r, sc):
        pltpu.async_copy(out_r, acc_sh.at[trg_big.at[j]], sc, add=True)

    def wait_sc(out_r, sc):
        pltpu.make_async_copy(out_r, acc_sh.at[trg_big.at[0]], sc).wait()

    def compute(j, a_r, st_r, out_r):
        jv = jnp.full((16,), j, jnp.int32)

        @plsc.parallel_loop(0, K, unroll=4)
        def edge(e):
            ev = jnp.full((16,), e, jnp.int32)
            pe = plsc.load_gather(p_big, [jv, ev])
            s = a_r[e, pl.ds(HF, 16)] + st_r[e, pl.ds(0, 16)] + pe * cvec
            s = jnp.where(s > 0, s, 0.2 * s)
            w = jnp.exp(s - mvec)
            plsc.store_scatter(out_r, [ev, widx], w, mask=wmask)
            for h in range(H):
                wh = plsc.load_gather(out_r, [ev, jnp.full((16,), HF + h,
                                                           jnp.int32)])
                out_r[e, pl.ds(h * 16, 16)] = a_r[e, pl.ds(h * 16, 16)] * wh

    def chunk(c, carry):
        @pl.when(c > 0)
        def _():
            wait_sc(out0, sem_sc0)
            wait_sc(out1, sem_sc1)

        r0 = wrb + c * CB
        pltpu.sync_copy(src_hbm.at[pl.ds(r0, CB)], src_big)
        pltpu.sync_copy(trg_hbm.at[pl.ds(r0, CB)], trg_big)
        pltpu.sync_copy(p_hbm.at[pl.ds(r0, CB)], p_big)
        issue(0, ax0, stx0, sem_ax0, sem_sx0)
        issue(1, ax1, stx1, sem_ax1, sem_sx1)

        def sup(q, carry2):
            j0 = 4 * q
            wait_g(ax0, stx0, sem_ax0, sem_sx0)
            wait_g(ax1, stx1, sem_ax1, sem_sx1)
            issue(j0 + 2, ay0, sty0, sem_ay0, sem_sy0)
            issue(j0 + 3, ay1, sty1, sem_ay1, sem_sy1)

            @pl.when(q >= 1)
            def _():
                wait_sc(out0, sem_sc0)
                wait_sc(out1, sem_sc1)

            compute(j0, ax0, stx0, out0)
            scat(j0, out0, sem_sc0)
            compute(j0 + 1, ax1, stx1, out1)
            scat(j0 + 1, out1, sem_sc1)
            wait_g(ay0, sty0, sem_ay0, sem_sy0)
            wait_g(ay1, sty1, sem_ay1, sem_sy1)
            issue(j0 + 4, ax0, stx0, sem_ax0, sem_sx0)
            issue(j0 + 5, ax1, stx1, sem_ax1, sem_sx1)
            wait_sc(out0, sem_sc0)
            wait_sc(out1, sem_sc1)
            compute(j0 + 2, ay0, sty0, out0)
            scat(j0 + 2, out0, sem_sc0)
            compute(j0 + 3, ay1, sty1, out1)
            scat(j0 + 3, out1, sem_sc1)
            return carry2

        lax.fori_loop(0, NSUP, sup, 0)
        wait_g(ax0, stx0, sem_ax0, sem_sx0)
        wait_g(ax1, stx1, sem_ax1, sem_sx1)
        wait_sc(out0, sem_sc0)
        wait_sc(out1, sem_sc1)
        compute(CB - 2, ax0, stx0, out0)
        scat(CB - 2, out0, sem_sc0)
        compute(CB - 1, ax1, stx1, out1)
        scat(CB - 1, out1, sem_sc1)
        return carry

    lax.fori_loop(0, NCH, chunk, 0)
    wait_sc(out0, sem_sc0)
    wait_sc(out1, sem_sc1)
    plsc.subcore_barrier()
    for jj in range(15):
        r0 = sid * RPS + jj * K
        pltpu.async_copy(acc_sh.at[pl.ds(r0, K)], acc_hbm.at[cid, pl.ds(r0, K)],
                         sem_sc0)
    r0 = sid * RPS + 600
    pltpu.async_copy(acc_sh.at[pl.ds(r0, 32)], acc_hbm.at[cid, pl.ds(r0, 32)],
                     sem_sc1)
    for jj in range(15):
        r0 = sid * RPS + jj * K
        pltpu.make_async_copy(acc_sh.at[pl.ds(r0, K)],
                              acc_hbm.at[cid, pl.ds(r0, K)], sem_sc0).wait()
    r0 = sid * RPS + 600
    pltpu.make_async_copy(acc_sh.at[pl.ds(r0, 32)],
                          acc_hbm.at[cid, pl.ds(r0, 32)], sem_sc1).wait()


_edge_sc = pl.kernel(
    _edge_body,
    out_type=jax.ShapeDtypeStruct((NC, NP, AW), jnp.float32),
    mesh=_SC_MESH,
    scratch_types=[
        pltpu.VMEM_SHARED((NP, AW), jnp.float32),
        pltpu.VMEM((2, 16), jnp.float32),
        pltpu.VMEM((CB, K), jnp.int32),
        pltpu.VMEM((CB, K), jnp.int32),
        pltpu.VMEM((CB, K), jnp.float32),
        pltpu.VMEM((K, RW), jnp.float32),
        pltpu.VMEM((K, RW), jnp.float32),
        pltpu.VMEM((K, RW), jnp.float32),
        pltpu.VMEM((K, RW), jnp.float32),
        pltpu.VMEM((K, 16), jnp.float32),
        pltpu.VMEM((K, 16), jnp.float32),
        pltpu.VMEM((K, 16), jnp.float32),
        pltpu.VMEM((K, 16), jnp.float32),
        pltpu.VMEM((K, AW), jnp.float32),
        pltpu.VMEM((K, AW), jnp.float32),
        pltpu.SemaphoreType.DMA,
        pltpu.SemaphoreType.DMA,
        pltpu.SemaphoreType.DMA,
        pltpu.SemaphoreType.DMA,
        pltpu.SemaphoreType.DMA,
        pltpu.SemaphoreType.DMA,
        pltpu.SemaphoreType.DMA,
        pltpu.SemaphoreType.DMA,
        pltpu.SemaphoreType.DMA,
        pltpu.SemaphoreType.DMA,
    ],
    compiler_params=pltpu.CompilerParams(use_tc_tiling_on_sc=False,
                                         needs_layout_passes=False),
)


def _edge_stage_jax(a_arr, st_arr, src, trg, p, c, m):
    proj = a_arr[:, :HF]
    ss = a_arr[:, HF:HF + H]
    st = st_arr[:, :H]
    s = ss[src] + st[trg] + p[:, None] * c[None, :]
    s = jnp.where(s > 0, s, 0.2 * s)
    w = jnp.exp(s - m[None, :])
    den = jax.ops.segment_sum(w, trg, num_segments=N)
    num = jax.ops.segment_sum(
        (w[:, :, None] * proj[src].reshape(E, H, F)).reshape(E, HF), trg,
        num_segments=N)
    part = jnp.concatenate([num, den, jnp.zeros((N, 8), jnp.float32)], axis=1)
    return jnp.stack([part, jnp.zeros_like(part)])


def kernel(x, edge_index, edge_prob, W_proj, W_tp, a_src, a_trg, a_tp, W_skip, bias):
    a_src128 = a_src.reshape(1, HF)
    a_trg128 = a_trg.reshape(1, HF)
    bias128 = bias.reshape(1, HF)
    a_arr, st_arr, bm = _pre(x, W_proj, a_src128, a_trg128)
    c = (W_tp[:, 0].reshape(H, F) * a_tp[0]).sum(-1)
    m = bm[:, :H].max(0) + bm[:, H:2 * H].max(0) + jnp.maximum(c, 0.0)
    src = edge_index[0]
    trg = edge_index[1]
    p = edge_prob[:, 0]
    cm = jnp.stack([
        jnp.concatenate([c, jnp.zeros((8,), jnp.float32)]),
        jnp.concatenate([m, jnp.full((8,), 1e9, jnp.float32)]),
    ])
    acc2 = _edge_sc(a_arr, st_arr, src.reshape(NROW, K), trg.reshape(NROW, K),
                    p.reshape(NROW, K), cm)
    out = _post(acc2, x, W_skip, bias128)
    return (out, edge_index, edge_prob)


# speedup vs baseline: 1.1972x; 1.0067x over previous
import functools

import jax
import jax.numpy as jnp
from jax import lax
from jax.experimental import pallas as pl
from jax.experimental.pallas import tpu as pltpu
from jax.experimental.pallas import tpu_sc as plsc

N, E, D, H, F = 10000, 320000, 128, 8, 16
HF = H * F
RW = HF + 16
BN = 2000
GRID = N // BN

_DOT = dict(preferred_element_type=jnp.float32, precision=lax.Precision.HIGHEST)


def _ehf():
    r = lax.broadcasted_iota(jnp.int32, (H, HF), 1) // F
    c = lax.broadcasted_iota(jnp.int32, (H, HF), 0)
    return (r == c).astype(jnp.float32)


def _pre_body(x_ref, wp_ref, asrc_ref, atrg_ref, a_ref, st_ref, bm_ref):
    x = x_ref[...]
    proj = lax.dot_general(x, wp_ref[...], (((1,), (1,)), ((), ())), **_DOT)
    ehf = _ehf()
    ss = lax.dot_general(proj * asrc_ref[...], ehf, (((1,), (1,)), ((), ())), **_DOT)
    st = lax.dot_general(proj * atrg_ref[...], ehf, (((1,), (1,)), ((), ())), **_DOT)
    zpad = jnp.zeros((x.shape[0], 8), jnp.float32)
    a_ref[...] = jnp.concatenate([proj, ss, zpad], axis=1)
    st_ref[...] = jnp.concatenate([st, zpad], axis=1)
    bm = jnp.concatenate([jnp.max(ss, axis=0), jnp.max(st, axis=0),
                          jnp.full((112,), -1e30, jnp.float32)])
    bm_ref[...] = jnp.broadcast_to(bm.reshape(1, HF), (8, HF))


def _pre(x, W_proj, a_src128, a_trg128):
    return pl.pallas_call(
        _pre_body,
        grid=(GRID,),
        in_specs=[
            pl.BlockSpec((BN, D), lambda i: (i, 0)),
            pl.BlockSpec((HF, D), lambda i: (0, 0)),
            pl.BlockSpec((1, HF), lambda i: (0, 0)),
            pl.BlockSpec((1, HF), lambda i: (0, 0)),
        ],
        out_specs=[
            pl.BlockSpec((BN, RW), lambda i: (i, 0)),
            pl.BlockSpec((BN, 16), lambda i: (i, 0)),
            pl.BlockSpec((8, HF), lambda i: (i, 0)),
        ],
        out_shape=[
            jax.ShapeDtypeStruct((N, RW), jnp.float32),
            jax.ShapeDtypeStruct((N, 16), jnp.float32),
            jax.ShapeDtypeStruct((GRID * 8, HF), jnp.float32),
        ],
    )(x, W_proj, a_src128, a_trg128)


def _post_body(acc_ref, x_ref, wsk_ref, bias_ref, out_ref):
    num = acc_ref[0, :, :HF] + acc_ref[1, :, :HF]
    den = acc_ref[0, :, HF:HF + H] + acc_ref[1, :, HF:HF + H]
    recip = 1.0 / (den + 1e-16)
    recip128 = lax.dot_general(recip, _ehf(), (((1,), (0,)), ((), ())), **_DOT)
    skip = lax.dot_general(x_ref[...], wsk_ref[...], (((1,), (1,)), ((), ())), **_DOT)
    o = num * recip128 + skip + bias_ref[...]
    out_ref[...] = jnp.where(o > 0, o, jnp.exp(jnp.minimum(o, 0.0)) - 1.0)


def _post(acc2, x, W_skip, bias128):
    return pl.pallas_call(
        _post_body,
        grid=(GRID,),
        in_specs=[
            pl.BlockSpec((2, BN, 136), lambda i: (0, i, 0)),
            pl.BlockSpec((BN, D), lambda i: (i, 0)),
            pl.BlockSpec((HF, D), lambda i: (0, 0)),
            pl.BlockSpec((1, HF), lambda i: (0, 0)),
        ],
        out_specs=pl.BlockSpec((BN, HF), lambda i: (i, 0)),
        out_shape=jax.ShapeDtypeStruct((N, HF), jnp.float32),
    )(acc2, x, W_skip, bias128)


NC, NS = 2, 16
NW = NC * NS
EPW = E // NW
K = 40
CB = 50
NCH = EPW // (K * CB)
NSUP = 12
NROW = E // K
RPW = EPW // K
AW = HF + H
NP = 10112
RPS = NP // NS
_SC_MESH = plsc.VectorSubcoreMesh(core_axis_name="c", subcore_axis_name="s")


def _edge_body(a_hbm, st_hbm, src_hbm, trg_hbm, p_hbm, cm_hbm, acc_hbm,
               acc_sh, cm_v, src_big, trg_big, p_big,
               ax0, ax1, ay0, ay1, stx0, stx1, sty0, sty1, out0, out1,
               sem_ax0, sem_ax1, sem_ay0, sem_ay1,
               sem_sx0, sem_sx1, sem_sy0, sem_sy1, sem_sc0, sem_sc1):
    cid = lax.axis_index("c")
    sid = lax.axis_index("s")
    zvec = jnp.zeros((16,), jnp.float32)

    def zrow(r, carry):
        for jj in range(HF // 16):
            out0[r, pl.ds(jj * 16, 16)] = zvec
        out0[r, pl.ds(AW - 16, 16)] = zvec
        return carry

    lax.fori_loop(0, K, zrow, 0)
    for jj in range(15):
        pltpu.async_copy(out0, acc_sh.at[pl.ds(sid * RPS + jj * K, K)], sem_sc0)
    pltpu.async_copy(out0.at[pl.ds(0, 32)],
                     acc_sh.at[pl.ds(sid * RPS + 600, 32)], sem_sc1)
    for jj in range(15):
        pltpu.make_async_copy(out0, acc_sh.at[pl.ds(sid * RPS + jj * K, K)],
                              sem_sc0).wait()
    pltpu.make_async_copy(out0.at[pl.ds(0, 32)],
                          acc_sh.at[pl.ds(sid * RPS + 600, 32)], sem_sc1).wait()
    pltpu.sync_copy(cm_hbm, cm_v)
    plsc.subcore_barrier()

    cvec = cm_v[0, :]
    mvec = cm_v[1, :]
    widx = jnp.arange(16, dtype=jnp.int32) + HF
    wmask = jnp.arange(16, dtype=jnp.int32) < H
    wrb = (cid * NS + sid) * RPW

    def issue(j, a_r, st_r, sa, ss_):
        pltpu.async_copy(a_hbm.at[src_big.at[j]], a_r, sa)
        pltpu.async_copy(st_hbm.at[trg_big.at[j]], st_r, ss_)

    def wait_g(a_r, st_r, sa, ss_):
        pltpu.make_async_copy(a_hbm.at[src_big.at[0]], a_r, sa).wait()
        pltpu.make_async_copy(st_hbm.at[trg_big.at[0]], st_r, ss_).wait()

    def scat(j, out_r, sc):
        pltpu.async_copy(out_r, acc_sh.at[trg_big.at[j]], sc, add=True)

    def wait_sc(out_r, sc):
        pltpu.make_async_copy(out_r, acc_sh.at[trg_big.at[0]], sc).wait()

    def compute(j, a_r, st_r, out_r):
        jv = jnp.full((16,), j, jnp.int32)

        @plsc.parallel_loop(0, K, unroll=4)
        def edge(e):
            ev = jnp.full((16,), e, jnp.int32)
            pe = plsc.load_gather(p_big, [jv, ev])
            s = a_r[e, pl.ds(HF, 16)] + st_r[e, pl.ds(0, 16)] + pe * cvec
            s = jnp.where(s > 0, s, 0.2 * s)
            w = jnp.exp(s - mvec)
            plsc.store_scatter(out_r, [ev, widx], w, mask=wmask)
            for h in range(H):
                wh = plsc.load_gather(out_r, [ev, jnp.full((16,), HF + h,
                                                           jnp.int32)])
                out_r[e, pl.ds(h * 16, 16)] = a_r[e, pl.ds(h * 16, 16)] * wh

    def chunk(c, carry):
        @pl.when(c > 0)
        def _():
            wait_sc(out0, sem_sc0)
            wait_sc(out1, sem_sc1)

        r0 = wrb + c * CB
        pltpu.sync_copy(src_hbm.at[pl.ds(r0, CB)], src_big)
        pltpu.sync_copy(trg_hbm.at[pl.ds(r0, CB)], trg_big)
        pltpu.sync_copy(p_hbm.at[pl.ds(r0, CB)], p_big)
        issue(0, ax0, stx0, sem_ax0, sem_sx0)
        issue(1, ax1, stx1, sem_ax1, sem_sx1)

        def sup(q, carry2):
            j0 = 4 * q
            issue(j0 + 2, ay0, sty0, sem_ay0, sem_sy0)
            issue(j0 + 3, ay1, sty1, sem_ay1, sem_sy1)
            wait_g(ax0, stx0, sem_ax0, sem_sx0)
            wait_g(ax1, stx1, sem_ax1, sem_sx1)

            @pl.when(q >= 1)
            def _():
                wait_sc(out0, sem_sc0)
                wait_sc(out1, sem_sc1)

            compute(j0, ax0, stx0, out0)
            scat(j0, out0, sem_sc0)
            compute(j0 + 1, ax1, stx1, out1)
            scat(j0 + 1, out1, sem_sc1)
            issue(j0 + 4, ax0, stx0, sem_ax0, sem_sx0)
            issue(j0 + 5, ax1, stx1, sem_ax1, sem_sx1)
            wait_g(ay0, sty0, sem_ay0, sem_sy0)
            wait_g(ay1, sty1, sem_ay1, sem_sy1)
            wait_sc(out0, sem_sc0)
            wait_sc(out1, sem_sc1)
            compute(j0 + 2, ay0, sty0, out0)
            scat(j0 + 2, out0, sem_sc0)
            compute(j0 + 3, ay1, sty1, out1)
            scat(j0 + 3, out1, sem_sc1)
            return carry2

        lax.fori_loop(0, NSUP, sup, 0)
        wait_g(ax0, stx0, sem_ax0, sem_sx0)
        wait_g(ax1, stx1, sem_ax1, sem_sx1)
        wait_sc(out0, sem_sc0)
        wait_sc(out1, sem_sc1)
        compute(CB - 2, ax0, stx0, out0)
        scat(CB - 2, out0, sem_sc0)
        compute(CB - 1, ax1, stx1, out1)
        scat(CB - 1, out1, sem_sc1)
        return carry

    lax.fori_loop(0, NCH, chunk, 0)
    wait_sc(out0, sem_sc0)
    wait_sc(out1, sem_sc1)
    plsc.subcore_barrier()
    for jj in range(15):
        r0 = sid * RPS + jj * K
        pltpu.async_copy(acc_sh.at[pl.ds(r0, K)], acc_hbm.at[cid, pl.ds(r0, K)],
                         sem_sc0)
    r0 = sid * RPS + 600
    pltpu.async_copy(acc_sh.at[pl.ds(r0, 32)], acc_hbm.at[cid, pl.ds(r0, 32)],
                     sem_sc1)
    for jj in range(15):
        r0 = sid * RPS + jj * K
        pltpu.make_async_copy(acc_sh.at[pl.ds(r0, K)],
                              acc_hbm.at[cid, pl.ds(r0, K)], sem_sc0).wait()
    r0 = sid * RPS + 600
    pltpu.make_async_copy(acc_sh.at[pl.ds(r0, 32)],
                          acc_hbm.at[cid, pl.ds(r0, 32)], sem_sc1).wait()


_edge_sc = pl.kernel(
    _edge_body,
    out_type=jax.ShapeDtypeStruct((NC, NP, AW), jnp.float32),
    mesh=_SC_MESH,
    scratch_types=[
        pltpu.VMEM_SHARED((NP, AW), jnp.float32),
        pltpu.VMEM((2, 16), jnp.float32),
        pltpu.VMEM((CB, K), jnp.int32),
        pltpu.VMEM((CB, K), jnp.int32),
        pltpu.VMEM((CB, K), jnp.float32),
        pltpu.VMEM((K, RW), jnp.float32),
        pltpu.VMEM((K, RW), jnp.float32),
        pltpu.VMEM((K, RW), jnp.float32),
        pltpu.VMEM((K, RW), jnp.float32),
        pltpu.VMEM((K, 16), jnp.float32),
        pltpu.VMEM((K, 16), jnp.float32),
        pltpu.VMEM((K, 16), jnp.float32),
        pltpu.VMEM((K, 16), jnp.float32),
        pltpu.VMEM((K, AW), jnp.float32),
        pltpu.VMEM((K, AW), jnp.float32),
        pltpu.SemaphoreType.DMA,
        pltpu.SemaphoreType.DMA,
        pltpu.SemaphoreType.DMA,
        pltpu.SemaphoreType.DMA,
        pltpu.SemaphoreType.DMA,
        pltpu.SemaphoreType.DMA,
        pltpu.SemaphoreType.DMA,
        pltpu.SemaphoreType.DMA,
        pltpu.SemaphoreType.DMA,
        pltpu.SemaphoreType.DMA,
    ],
    compiler_params=pltpu.CompilerParams(use_tc_tiling_on_sc=False,
                                         needs_layout_passes=False),
)


def _edge_stage_jax(a_arr, st_arr, src, trg, p, c, m):
    proj = a_arr[:, :HF]
    ss = a_arr[:, HF:HF + H]
    st = st_arr[:, :H]
    s = ss[src] + st[trg] + p[:, None] * c[None, :]
    s = jnp.where(s > 0, s, 0.2 * s)
    w = jnp.exp(s - m[None, :])
    den = jax.ops.segment_sum(w, trg, num_segments=N)
    num = jax.ops.segment_sum(
        (w[:, :, None] * proj[src].reshape(E, H, F)).reshape(E, HF), trg,
        num_segments=N)
    part = jnp.concatenate([num, den, jnp.zeros((N, 8), jnp.float32)], axis=1)
    return jnp.stack([part, jnp.zeros_like(part)])


def kernel(x, edge_index, edge_prob, W_proj, W_tp, a_src, a_trg, a_tp, W_skip, bias):
    a_src128 = a_src.reshape(1, HF)
    a_trg128 = a_trg.reshape(1, HF)
    bias128 = bias.reshape(1, HF)
    a_arr, st_arr, bm = _pre(x, W_proj, a_src128, a_trg128)
    c = (W_tp[:, 0].reshape(H, F) * a_tp[0]).sum(-1)
    m = bm[:, :H].max(0) + bm[:, H:2 * H].max(0) + jnp.maximum(c, 0.0)
    src = edge_index[0]
    trg = edge_index[1]
    p = edge_prob[:, 0]
    cm = jnp.stack([
        jnp.concatenate([c, jnp.zeros((8,), jnp.float32)]),
        jnp.concatenate([m, jnp.full((8,), 1e9, jnp.float32)]),
    ])
    acc2 = _edge_sc(a_arr, st_arr, src.reshape(NROW, K), trg.reshape(NROW, K),
                    p.reshape(NROW, K), cm)
    out = _post(acc2, x, W_skip, bias128)
    return (out, edge_index, edge_prob)


# speedup vs baseline: 1.3813x; 1.1538x over previous
import functools

import jax
import jax.numpy as jnp
from jax import lax
from jax.experimental import pallas as pl
from jax.experimental.pallas import tpu as pltpu
from jax.experimental.pallas import tpu_sc as plsc

N, E, D, H, F = 10000, 320000, 128, 8, 16
HF = H * F
RW = HF + 16
AT = 80
BN = 2000
GRID = N // BN

_DOT = dict(preferred_element_type=jnp.float32, precision=lax.Precision.HIGHEST)


def _ehf():
    r = lax.broadcasted_iota(jnp.int32, (H, HF), 1) // F
    c = lax.broadcasted_iota(jnp.int32, (H, HF), 0)
    return (r == c).astype(jnp.float32)


def _pre_body(x_ref, wp_ref, asrc_ref, atrg_ref, a_ref, st_ref, bm_ref):
    x = x_ref[...]
    proj = lax.dot_general(x, wp_ref[...], (((1,), (1,)), ((), ())), **_DOT)
    ehf = _ehf()
    ss = lax.dot_general(proj * asrc_ref[...], ehf, (((1,), (1,)), ((), ())), **_DOT)
    st = lax.dot_general(proj * atrg_ref[...], ehf, (((1,), (1,)), ((), ())), **_DOT)
    zpad = jnp.zeros((x.shape[0], 8), jnp.float32)
    u = lax.bitcast_convert_type(proj.astype(jnp.bfloat16), jnp.uint16)
    lo = u[:, :64].astype(jnp.uint32)
    hi = u[:, 64:].astype(jnp.uint32)
    p32 = lo | (hi << jnp.uint32(16))
    ssu = lax.bitcast_convert_type(ss, jnp.uint32)
    zpu = jnp.zeros((x.shape[0], 8), jnp.uint32)
    a_ref[...] = jnp.concatenate([p32, ssu, zpu], axis=1)
    st_ref[...] = jnp.concatenate([st, zpad], axis=1)
    bm = jnp.concatenate([jnp.max(ss, axis=0), jnp.max(st, axis=0),
                          jnp.full((112,), -1e30, jnp.float32)])
    bm_ref[...] = jnp.broadcast_to(bm.reshape(1, HF), (8, HF))


def _pre(x, W_proj, a_src128, a_trg128):
    return pl.pallas_call(
        _pre_body,
        grid=(GRID,),
        in_specs=[
            pl.BlockSpec((BN, D), lambda i: (i, 0)),
            pl.BlockSpec((HF, D), lambda i: (0, 0)),
            pl.BlockSpec((1, HF), lambda i: (0, 0)),
            pl.BlockSpec((1, HF), lambda i: (0, 0)),
        ],
        out_specs=[
            pl.BlockSpec((BN, AT), lambda i: (i, 0)),
            pl.BlockSpec((BN, 16), lambda i: (i, 0)),
            pl.BlockSpec((8, HF), lambda i: (i, 0)),
        ],
        out_shape=[
            jax.ShapeDtypeStruct((N, AT), jnp.uint32),
            jax.ShapeDtypeStruct((N, 16), jnp.float32),
            jax.ShapeDtypeStruct((GRID * 8, HF), jnp.float32),
        ],
    )(x, W_proj, a_src128, a_trg128)


def _post_body(acc_ref, x_ref, wsk_ref, bias_ref, out_ref):
    num = acc_ref[0, :, :HF] + acc_ref[1, :, :HF]
    den = acc_ref[0, :, HF:HF + H] + acc_ref[1, :, HF:HF + H]
    recip = 1.0 / (den + 1e-16)
    recip128 = lax.dot_general(recip, _ehf(), (((1,), (0,)), ((), ())), **_DOT)
    skip = lax.dot_general(x_ref[...], wsk_ref[...], (((1,), (1,)), ((), ())), **_DOT)
    o = num * recip128 + skip + bias_ref[...]
    out_ref[...] = jnp.where(o > 0, o, jnp.exp(jnp.minimum(o, 0.0)) - 1.0)


def _post(acc2, x, W_skip, bias128):
    return pl.pallas_call(
        _post_body,
        grid=(GRID,),
        in_specs=[
            pl.BlockSpec((2, BN, 136), lambda i: (0, i, 0)),
            pl.BlockSpec((BN, D), lambda i: (i, 0)),
            pl.BlockSpec((HF, D), lambda i: (0, 0)),
            pl.BlockSpec((1, HF), lambda i: (0, 0)),
        ],
        out_specs=pl.BlockSpec((BN, HF), lambda i: (i, 0)),
        out_shape=jax.ShapeDtypeStruct((N, HF), jnp.float32),
    )(acc2, x, W_skip, bias128)


NC, NS = 2, 16
NW = NC * NS
EPW = E // NW
K = 40
CB = 50
NCH = EPW // (K * CB)
NSUP = 12
NROW = E // K
RPW = EPW // K
AW = HF + H
NP = 10112
RPS = NP // NS
_SC_MESH = plsc.VectorSubcoreMesh(core_axis_name="c", subcore_axis_name="s")


def _edge_body(a_hbm, st_hbm, src_hbm, trg_hbm, p_hbm, cm_hbm, acc_hbm,
               acc_sh, cm_v, src_big, trg_big, p_big,
               ax0, ax1, ay0, ay1, stx0, stx1, sty0, sty1, out0, out1,
               sem_ax0, sem_ax1, sem_ay0, sem_ay1,
               sem_sx0, sem_sx1, sem_sy0, sem_sy1, sem_sc0, sem_sc1):
    cid = lax.axis_index("c")
    sid = lax.axis_index("s")
    zvec = jnp.zeros((16,), jnp.float32)

    def zrow(r, carry):
        for jj in range(HF // 16):
            out0[r, pl.ds(jj * 16, 16)] = zvec
        out0[r, pl.ds(AW - 16, 16)] = zvec
        return carry

    lax.fori_loop(0, K, zrow, 0)
    for jj in range(15):
        pltpu.async_copy(out0, acc_sh.at[pl.ds(sid * RPS + jj * K, K)], sem_sc0)
    pltpu.async_copy(out0.at[pl.ds(0, 32)],
                     acc_sh.at[pl.ds(sid * RPS + 600, 32)], sem_sc1)
    for jj in range(15):
        pltpu.make_async_copy(out0, acc_sh.at[pl.ds(sid * RPS + jj * K, K)],
                              sem_sc0).wait()
    pltpu.make_async_copy(out0.at[pl.ds(0, 32)],
                          acc_sh.at[pl.ds(sid * RPS + 600, 32)], sem_sc1).wait()
    pltpu.sync_copy(cm_hbm, cm_v)
    plsc.subcore_barrier()

    cvec = cm_v[0, :]
    mvec = cm_v[1, :]
    widx = jnp.arange(16, dtype=jnp.int32) + HF
    wmask = jnp.arange(16, dtype=jnp.int32) < H
    wrb = (cid * NS + sid) * RPW

    def issue(j, a_r, st_r, sa, ss_):
        pltpu.async_copy(a_hbm.at[src_big.at[j]], a_r, sa)
        pltpu.async_copy(st_hbm.at[trg_big.at[j]], st_r, ss_)

    def wait_g(a_r, st_r, sa, ss_):
        pltpu.make_async_copy(a_hbm.at[src_big.at[0]], a_r, sa).wait()
        pltpu.make_async_copy(st_hbm.at[trg_big.at[0]], st_r, ss_).wait()

    def scat(j, out_r, sc):
        pltpu.async_copy(out_r, acc_sh.at[trg_big.at[j]], sc, add=True)

    def wait_sc(out_r, sc):
        pltpu.make_async_copy(out_r, acc_sh.at[trg_big.at[0]], sc).wait()

    def compute(j, a_r, st_r, out_r):
        jv = jnp.full((16,), j, jnp.int32)

        @plsc.parallel_loop(0, K, unroll=4)
        def edge(e):
            ev = jnp.full((16,), e, jnp.int32)
            pe = plsc.load_gather(p_big, [jv, ev])
            ssv = plsc.bitcast(a_r[e, pl.ds(64, 16)], jnp.float32)
            s = ssv + st_r[e, pl.ds(0, 16)] + pe * cvec
            s = jnp.where(s > 0, s, 0.2 * s)
            w = jnp.exp(s - mvec)
            plsc.store_scatter(out_r, [ev, widx], w, mask=wmask)
            for t in range(4):
                v = a_r[e, pl.ds(16 * t, 16)]
                lov = plsc.bitcast(v << jnp.uint32(16), jnp.float32)
                hiv = plsc.bitcast(v & jnp.uint32(0xFFFF0000), jnp.float32)
                out_r[e, pl.ds(t * 16, 16)] = lov * w[t]
                out_r[e, pl.ds((4 + t) * 16, 16)] = hiv * w[4 + t]

    def chunk(c, carry):
        @pl.when(c > 0)
        def _():
            wait_sc(out0, sem_sc0)
            wait_sc(out1, sem_sc1)

        r0 = wrb + c * CB
        pltpu.sync_copy(src_hbm.at[pl.ds(r0, CB)], src_big)
        pltpu.sync_copy(trg_hbm.at[pl.ds(r0, CB)], trg_big)
        pltpu.sync_copy(p_hbm.at[pl.ds(r0, CB)], p_big)
        issue(0, ax0, stx0, sem_ax0, sem_sx0)
        issue(1, ax1, stx1, sem_ax1, sem_sx1)

        def sup(q, carry2):
            j0 = 4 * q
            issue(j0 + 2, ay0, sty0, sem_ay0, sem_sy0)
            issue(j0 + 3, ay1, sty1, sem_ay1, sem_sy1)
            wait_g(ax0, stx0, sem_ax0, sem_sx0)
            wait_g(ax1, stx1, sem_ax1, sem_sx1)

            @pl.when(q >= 1)
            def _():
                wait_sc(out0, sem_sc0)
                wait_sc(out1, sem_sc1)

            compute(j0, ax0, stx0, out0)
            scat(j0, out0, sem_sc0)
            compute(j0 + 1, ax1, stx1, out1)
            scat(j0 + 1, out1, sem_sc1)
            issue(j0 + 4, ax0, stx0, sem_ax0, sem_sx0)
            issue(j0 + 5, ax1, stx1, sem_ax1, sem_sx1)
            wait_g(ay0, sty0, sem_ay0, sem_sy0)
            wait_g(ay1, sty1, sem_ay1, sem_sy1)
            wait_sc(out0, sem_sc0)
            wait_sc(out1, sem_sc1)
            compute(j0 + 2, ay0, sty0, out0)
            scat(j0 + 2, out0, sem_sc0)
            compute(j0 + 3, ay1, sty1, out1)
            scat(j0 + 3, out1, sem_sc1)
            return carry2

        lax.fori_loop(0, NSUP, sup, 0)
        wait_g(ax0, stx0, sem_ax0, sem_sx0)
        wait_g(ax1, stx1, sem_ax1, sem_sx1)
        wait_sc(out0, sem_sc0)
        wait_sc(out1, sem_sc1)
        compute(CB - 2, ax0, stx0, out0)
        scat(CB - 2, out0, sem_sc0)
        compute(CB - 1, ax1, stx1, out1)
        scat(CB - 1, out1, sem_sc1)
        return carry

    lax.fori_loop(0, NCH, chunk, 0)
    wait_sc(out0, sem_sc0)
    wait_sc(out1, sem_sc1)
    plsc.subcore_barrier()
    for jj in range(15):
        r0 = sid * RPS + jj * K
        pltpu.async_copy(acc_sh.at[pl.ds(r0, K)], acc_hbm.at[cid, pl.ds(r0, K)],
                         sem_sc0)
    r0 = sid * RPS + 600
    pltpu.async_copy(acc_sh.at[pl.ds(r0, 32)], acc_hbm.at[cid, pl.ds(r0, 32)],
                     sem_sc1)
    for jj in range(15):
        r0 = sid * RPS + jj * K
        pltpu.make_async_copy(acc_sh.at[pl.ds(r0, K)],
                              acc_hbm.at[cid, pl.ds(r0, K)], sem_sc0).wait()
    r0 = sid * RPS + 600
    pltpu.make_async_copy(acc_sh.at[pl.ds(r0, 32)],
                          acc_hbm.at[cid, pl.ds(r0, 32)], sem_sc1).wait()


_edge_sc = pl.kernel(
    _edge_body,
    out_type=jax.ShapeDtypeStruct((NC, NP, AW), jnp.float32),
    mesh=_SC_MESH,
    scratch_types=[
        pltpu.VMEM_SHARED((NP, AW), jnp.float32),
        pltpu.VMEM((2, 16), jnp.float32),
        pltpu.VMEM((CB, K), jnp.int32),
        pltpu.VMEM((CB, K), jnp.int32),
        pltpu.VMEM((CB, K), jnp.float32),
        pltpu.VMEM((K, AT), jnp.uint32),
        pltpu.VMEM((K, AT), jnp.uint32),
        pltpu.VMEM((K, AT), jnp.uint32),
        pltpu.VMEM((K, AT), jnp.uint32),
        pltpu.VMEM((K, 16), jnp.float32),
        pltpu.VMEM((K, 16), jnp.float32),
        pltpu.VMEM((K, 16), jnp.float32),
        pltpu.VMEM((K, 16), jnp.float32),
        pltpu.VMEM((K, AW), jnp.float32),
        pltpu.VMEM((K, AW), jnp.float32),
        pltpu.SemaphoreType.DMA,
        pltpu.SemaphoreType.DMA,
        pltpu.SemaphoreType.DMA,
        pltpu.SemaphoreType.DMA,
        pltpu.SemaphoreType.DMA,
        pltpu.SemaphoreType.DMA,
        pltpu.SemaphoreType.DMA,
        pltpu.SemaphoreType.DMA,
        pltpu.SemaphoreType.DMA,
        pltpu.SemaphoreType.DMA,
    ],
    compiler_params=pltpu.CompilerParams(use_tc_tiling_on_sc=False,
                                         needs_layout_passes=False),
)


def _edge_stage_jax(a_arr, st_arr, src, trg, p, c, m):
    proj = a_arr[:, :HF]
    ss = a_arr[:, HF:HF + H]
    st = st_arr[:, :H]
    s = ss[src] + st[trg] + p[:, None] * c[None, :]
    s = jnp.where(s > 0, s, 0.2 * s)
    w = jnp.exp(s - m[None, :])
    den = jax.ops.segment_sum(w, trg, num_segments=N)
    num = jax.ops.segment_sum(
        (w[:, :, None] * proj[src].reshape(E, H, F)).reshape(E, HF), trg,
        num_segments=N)
    part = jnp.concatenate([num, den, jnp.zeros((N, 8), jnp.float32)], axis=1)
    return jnp.stack([part, jnp.zeros_like(part)])


def kernel(x, edge_index, edge_prob, W_proj, W_tp, a_src, a_trg, a_tp, W_skip, bias):
    a_src128 = a_src.reshape(1, HF)
    a_trg128 = a_trg.reshape(1, HF)
    bias128 = bias.reshape(1, HF)
    a_arr, st_arr, bm = _pre(x, W_proj, a_src128, a_trg128)
    c = (W_tp[:, 0].reshape(H, F) * a_tp[0]).sum(-1)
    m = bm[:, :H].max(0) + bm[:, H:2 * H].max(0) + jnp.maximum(c, 0.0)
    src = edge_index[0]
    trg = edge_index[1]
    p = edge_prob[:, 0]
    cm = jnp.stack([
        jnp.concatenate([c, jnp.zeros((8,), jnp.float32)]),
        jnp.concatenate([m, jnp.full((8,), 1e9, jnp.float32)]),
    ])
    acc2 = _edge_sc(a_arr, st_arr, src.reshape(NROW, K), trg.reshape(NROW, K),
                    p.reshape(NROW, K), cm)
    out = _post(acc2, x, W_skip, bias128)
    return (out, edge_index, edge_prob)
